# Initial kernel scaffold; baseline (speedup 1.0000x reference)
import jax
import jax.numpy as jnp
from jax.experimental import pallas as pl


def kernel(degree, degree_encoder_weight):
    raise NotImplementedError("write your pallas kernel here")


# speedup vs baseline: 3.5867x; 3.5867x over previous
import functools

import jax
import jax.numpy as jnp
from jax import lax
from jax.experimental import pallas as pl
from jax.experimental.pallas import tpu as pltpu
from jax.experimental.pallas import tpu_sc as plsc

NUM_DEGREE = 1000
D_MODEL = 64
B_TOTAL = 4096 * 200
NC, NS = 2, 16
NW = NC * NS
B_PER_W = B_TOTAL // NW
SUB = 128
CHUNK = 512
N_SUB = CHUNK // SUB
N_CHUNKS = B_PER_W // CHUNK
IDX_ROWS_PER_W = B_PER_W // SUB


@functools.partial(
    pl.kernel,
    out_type=jax.ShapeDtypeStruct((B_TOTAL, D_MODEL), jnp.float32),
    mesh=plsc.VectorSubcoreMesh(core_axis_name="c", subcore_axis_name="s"),
    compiler_params=pltpu.CompilerParams(use_tc_tiling_on_sc=False),
    scratch_types=[
        pltpu.VMEM((N_SUB, SUB), jnp.int32),
        pltpu.VMEM((CHUNK, D_MODEL), jnp.float32),
        pltpu.SemaphoreType.DMA,
    ],
)
def _gather_kernel(idx_hbm, table_hbm, out_hbm, idx_v, rows_v, sem):
    wid = lax.axis_index("s") * NC + lax.axis_index("c")
    idx_row_base = wid * IDX_ROWS_PER_W
    out_base = wid * B_PER_W

    def chunk_body(ci, carry):
        pltpu.sync_copy(
            idx_hbm.at[pl.ds(idx_row_base + ci * N_SUB, N_SUB)], idx_v
        )
        copies = [
            pltpu.async_copy(
                table_hbm.at[idx_v.at[j]],
                rows_v.at[pl.ds(j * SUB, SUB)],
                sem,
            )
            for j in range(N_SUB)
        ]
        for c in copies:
            c.wait()
        pltpu.sync_copy(
            rows_v, out_hbm.at[pl.ds(out_base + ci * CHUNK, CHUNK)]
        )
        return carry

    lax.fori_loop(0, N_CHUNKS, chunk_body, 0)


def kernel(degree, degree_encoder_weight):
    idx2d = degree.reshape(B_TOTAL // SUB, SUB)
    out = _gather_kernel(idx2d, degree_encoder_weight)
    return out.reshape(degree.shape[0], degree.shape[1], D_MODEL)
